# baseline (speedup 1.0000x reference)
import jax
import jax.numpy as jnp
from jax.experimental import pallas as pl

_B, _H, _KV, _Q, _DH = 8, 16, 2048, 16, 128
_BH = _B * _H


def _concat_copy(pk_ref, pv_ref, ks_ref, vs_ref, ok_ref, ov_ref):
    ok_ref[0, : _KV, :] = pk_ref[0]
    ok_ref[0, _KV :, :] = ks_ref[0]
    ov_ref[0, : _KV, :] = pv_ref[0]
    ov_ref[0, _KV :, :] = vs_ref[0]


def kernel(past_key, past_value, key_states, value_states, layer_idx):
    pk = past_key.reshape(_BH, _KV, _DH)
    pv = past_value.reshape(_BH, _KV, _DH)
    ks = key_states.reshape(_BH, _Q, _DH)
    vs = value_states.reshape(_BH, _Q, _DH)

    big_spec = pl.BlockSpec((1, _KV, _DH), lambda i: (i, 0, 0))
    small_spec = pl.BlockSpec((1, _Q, _DH), lambda i: (i, 0, 0))
    out_spec = pl.BlockSpec((1, _KV + _Q, _DH), lambda i: (i, 0, 0))
    out_shape = jax.ShapeDtypeStruct((_BH, _KV + _Q, _DH), jnp.float32)

    ok, ov = pl.pallas_call(
        _concat_copy,
        grid=(_BH,),
        in_specs=[big_spec, big_spec, small_spec, small_spec],
        out_specs=[out_spec, out_spec],
        out_shape=[out_shape, out_shape],
    )(pk, pv, ks, vs)

    ok = ok.reshape(_B, _H, _KV + _Q, _DH)
    ov = ov.reshape(_B, _H, _KV + _Q, _DH)
    return (ok, ov)
